# baseline (speedup 1.0000x reference)
import functools

import jax
import jax.numpy as jnp
from jax import lax
from jax.experimental import pallas as pl
from jax.experimental.pallas import tpu as pltpu
from jax.experimental.pallas import tpu_sc as plsc

B, S, H, E, K = 4, 4096, 768, 64, 2
T = B * S
BM = 128
R = T * K + E * BM
NBLK = R // BM
BT = 1024

NC, NS = 2, 16
NW = NC * NS
GCH = 64


def _router_body(x_ref, gwt_ref, gb_ref, logits_ref, idx_ref, w_ref):
    xb = x_ref[...]
    logits = lax.dot_general(xb, gwt_ref[...], (((1,), (0,)), ((), ())),
                             preferred_element_type=jnp.float32)
    logits = logits + gb_ref[...]
    logits_ref[...] = logits
    ids = lax.broadcasted_iota(jnp.int32, logits.shape, 1)
    m1 = jnp.max(logits, axis=1, keepdims=True)
    i1 = jnp.min(jnp.where(logits == m1, ids, E), axis=1, keepdims=True)
    masked = jnp.where(ids == i1, -jnp.inf, logits)
    m2 = jnp.max(masked, axis=1, keepdims=True)
    i2 = jnp.min(jnp.where(masked == m2, ids, E), axis=1, keepdims=True)
    w1 = 1.0 / (1.0 + jnp.exp(m2 - m1))
    idx_ref[...] = jnp.concatenate([i1, i2], axis=1)
    w_ref[...] = jnp.concatenate([w1, 1.0 - w1], axis=1)


def _router(hidden, gate_w, gate_b):
    return pl.pallas_call(
        _router_body,
        grid=(T // BT,),
        in_specs=[
            pl.BlockSpec((BT, H), lambda i: (i, 0)),
            pl.BlockSpec((H, E), lambda i: (0, 0)),
            pl.BlockSpec((1, E), lambda i: (0, 0)),
        ],
        out_specs=[
            pl.BlockSpec((BT, E), lambda i: (i, 0)),
            pl.BlockSpec((BT, K), lambda i: (i, 0)),
            pl.BlockSpec((BT, K), lambda i: (i, 0)),
        ],
        out_shape=[
            jax.ShapeDtypeStruct((T, E), jnp.float32),
            jax.ShapeDtypeStruct((T, K), jnp.int32),
            jax.ShapeDtypeStruct((T, K), jnp.float32),
        ],
    )(hidden, gate_w.T, gate_b.reshape(1, E))


def _sc_worker_id():
    return lax.axis_index("s") * NC + lax.axis_index("c")


@functools.partial(
    pl.kernel,
    out_type=jax.ShapeDtypeStruct((R, H), jnp.float32),
    mesh=plsc.VectorSubcoreMesh(core_axis_name="c", subcore_axis_name="s"),
    scratch_types=[
        pltpu.VMEM((GCH,), jnp.int32),
        pltpu.VMEM((GCH, H), jnp.float32),
        pltpu.SemaphoreType.DMA,
    ],
)
def _sc_gather_rows(hid_hbm, src_hbm, out_hbm, idx_v, rows_v, sem):
    base = _sc_worker_id() * (R // NW)
    def body(i, carry):
        b = base + i * GCH
        pltpu.sync_copy(src_hbm.at[pl.ds(b, GCH)], idx_v)
        pltpu.async_copy(hid_hbm.at[idx_v], rows_v, sem).wait()
        pltpu.sync_copy(rows_v, out_hbm.at[pl.ds(b, GCH)])
        return carry
    lax.fori_loop(0, (R // NW) // GCH, body, 0)


def _gemm_body(be_ref, xs_ref, ew_ref, eb_ref, ws_ref, out_ref):
    acc = lax.dot_general(xs_ref[...], ew_ref[0], (((1,), (1,)), ((), ())),
                          preferred_element_type=jnp.float32)
    out_ref[...] = (acc + eb_ref[0]) * ws_ref[...]


def _grouped_gemm(x_sorted, expert_w, expert_b, w_sorted, blk_e):
    grid_spec = pltpu.PrefetchScalarGridSpec(
        num_scalar_prefetch=1,
        grid=(NBLK,),
        in_specs=[
            pl.BlockSpec((BM, H), lambda i, be: (i, 0)),
            pl.BlockSpec((1, H, H), lambda i, be: (be[i], 0, 0)),
            pl.BlockSpec((1, 1, H), lambda i, be: (be[i], 0, 0)),
            pl.BlockSpec((BM, 1), lambda i, be: (i, 0)),
        ],
        out_specs=pl.BlockSpec((BM, H), lambda i, be: (i, 0)),
    )
    return pl.pallas_call(
        _gemm_body,
        grid_spec=grid_spec,
        out_shape=jax.ShapeDtypeStruct((R, H), jnp.float32),
    )(blk_e, x_sorted, expert_w, expert_b.reshape(E, 1, H),
      w_sorted.reshape(R, 1))


@functools.partial(
    pl.kernel,
    out_type=(jax.ShapeDtypeStruct((T, H), jnp.float32),
              jax.ShapeDtypeStruct((T, H), jnp.float32)),
    mesh=plsc.VectorSubcoreMesh(core_axis_name="c", subcore_axis_name="s"),
    scratch_types=[
        pltpu.VMEM((GCH,), jnp.int32),
        pltpu.VMEM((GCH,), jnp.int32),
        pltpu.VMEM((GCH, H), jnp.float32),
        pltpu.VMEM((GCH, H), jnp.float32),
        pltpu.SemaphoreType.DMA,
        pltpu.SemaphoreType.DMA,
    ],
)
def _sc_gather_pairs(outs_hbm, p0_hbm, p1_hbm, g0_hbm, g1_hbm,
                     i0_v, i1_v, b0_v, b1_v, sem0, sem1):
    base = _sc_worker_id() * (T // NW)
    def body(i, carry):
        b = base + i * GCH
        pltpu.sync_copy(p0_hbm.at[pl.ds(b, GCH)], i0_v)
        pltpu.sync_copy(p1_hbm.at[pl.ds(b, GCH)], i1_v)
        c0 = pltpu.async_copy(outs_hbm.at[i0_v], b0_v, sem0)
        c1 = pltpu.async_copy(outs_hbm.at[i1_v], b1_v, sem1)
        c0.wait()
        c1.wait()
        pltpu.sync_copy(b0_v, g0_hbm.at[pl.ds(b, GCH)])
        pltpu.sync_copy(b1_v, g1_hbm.at[pl.ds(b, GCH)])
        return carry
    lax.fori_loop(0, (T // NW) // GCH, body, 0)


def _add_body(a_ref, b_ref, o_ref):
    o_ref[...] = a_ref[...] + b_ref[...]


def _combine_add(g0, g1):
    return pl.pallas_call(
        _add_body,
        grid=(T // BT,),
        in_specs=[pl.BlockSpec((BT, H), lambda i: (i, 0)),
                  pl.BlockSpec((BT, H), lambda i: (i, 0))],
        out_specs=pl.BlockSpec((BT, H), lambda i: (i, 0)),
        out_shape=jax.ShapeDtypeStruct((T, H), jnp.float32),
    )(g0, g1)


def kernel(x, gate_w, gate_b, expert_w, expert_b):
    hidden = x.reshape(T, H)
    logits, top_idx, top_w = _router(hidden, gate_w, gate_b)

    flat_e = top_idx.reshape(-1)
    flat_w = top_w.reshape(-1)
    perm = jnp.argsort(flat_e, stable=True).astype(jnp.int32)
    sorted_e = flat_e[perm]
    counts = jnp.zeros((E,), jnp.int32).at[flat_e].add(1)
    padded = ((counts + BM - 1) // BM) * BM
    cpad = jnp.cumsum(padded)
    poff = cpad - padded
    coff = jnp.cumsum(counts) - counts
    j = jnp.arange(T * K, dtype=jnp.int32)
    dest = (poff[sorted_e] + j - coff[sorted_e]).astype(jnp.int32)
    src_row = jnp.zeros((R,), jnp.int32).at[dest].set(perm // K)
    w_sorted = jnp.zeros((R,), jnp.float32).at[dest].set(flat_w[perm])
    pos = jnp.zeros((T * K,), jnp.int32).at[perm].set(dest)
    p0 = pos[0::2]
    p1 = pos[1::2]
    blk_start = jnp.arange(NBLK, dtype=jnp.int32) * BM
    blk_e = jnp.minimum(
        jnp.searchsorted(cpad, blk_start, side="right").astype(jnp.int32),
        E - 1)

    x_sorted = jnp.take(hidden, src_row, axis=0)
    out_sorted = _grouped_gemm(x_sorted, expert_w, expert_b, w_sorted, blk_e)
    g0, g1 = _sc_gather_pairs(out_sorted, p0, p1)
    final = _combine_add(g0, g1)
    return final.reshape(B, S, H), logits


# speedup vs baseline: 1.0423x; 1.0423x over previous
import functools

import jax
import jax.numpy as jnp
from jax import lax
from jax.experimental import pallas as pl
from jax.experimental.pallas import tpu as pltpu
from jax.experimental.pallas import tpu_sc as plsc

B, S, H, E, K = 4, 4096, 768, 64, 2
T = B * S
BM = 128
R = T * K + E * BM
NBLK = R // BM
BT = 1024

NC, NS = 2, 16
NW = NC * NS
GCH = 64


def _router_body(x_ref, gwt_ref, gb_ref, logits_ref, idx_ref, w_ref):
    xb = x_ref[...]
    logits = lax.dot_general(xb, gwt_ref[...], (((1,), (0,)), ((), ())),
                             preferred_element_type=jnp.float32)
    logits = logits + gb_ref[...]
    logits_ref[...] = logits
    ids = lax.broadcasted_iota(jnp.int32, logits.shape, 1)
    m1 = jnp.max(logits, axis=1, keepdims=True)
    i1 = jnp.min(jnp.where(logits == m1, ids, E), axis=1, keepdims=True)
    masked = jnp.where(ids == i1, -jnp.inf, logits)
    m2 = jnp.max(masked, axis=1, keepdims=True)
    i2 = jnp.min(jnp.where(masked == m2, ids, E), axis=1, keepdims=True)
    w1 = 1.0 / (1.0 + jnp.exp(m2 - m1))
    idx_ref[...] = jnp.concatenate([i1, i2], axis=1)
    w_ref[...] = jnp.concatenate([w1, 1.0 - w1], axis=1)


def _router(hidden, gate_w, gate_b):
    return pl.pallas_call(
        _router_body,
        grid=(T // BT,),
        in_specs=[
            pl.BlockSpec((BT, H), lambda i: (i, 0)),
            pl.BlockSpec((H, E), lambda i: (0, 0)),
            pl.BlockSpec((1, E), lambda i: (0, 0)),
        ],
        out_specs=[
            pl.BlockSpec((BT, E), lambda i: (i, 0)),
            pl.BlockSpec((BT, K), lambda i: (i, 0)),
            pl.BlockSpec((BT, K), lambda i: (i, 0)),
        ],
        out_shape=[
            jax.ShapeDtypeStruct((T, E), jnp.float32),
            jax.ShapeDtypeStruct((T, K), jnp.int32),
            jax.ShapeDtypeStruct((T, K), jnp.float32),
        ],
    )(hidden, gate_w.T, gate_b.reshape(1, E))


def _sc_worker_id():
    return lax.axis_index("s") * NC + lax.axis_index("c")


@functools.partial(
    pl.kernel,
    out_type=jax.ShapeDtypeStruct((R, H), jnp.float32),
    mesh=plsc.VectorSubcoreMesh(core_axis_name="c", subcore_axis_name="s"),
    scratch_types=[
        pltpu.VMEM((GCH,), jnp.int32),
        pltpu.VMEM((GCH,), jnp.int32),
        pltpu.VMEM((GCH, H), jnp.float32),
        pltpu.VMEM((GCH, H), jnp.float32),
        pltpu.SemaphoreType.DMA,
        pltpu.SemaphoreType.DMA,
    ],
)
def _sc_gather_rows(hid_hbm, src_hbm, out_hbm, i0_v, i1_v, b0_v, b1_v,
                    sem0, sem1):
    base = _sc_worker_id() * (R // NW)
    def body(i, carry):
        b = base + i * (2 * GCH)
        pltpu.sync_copy(src_hbm.at[pl.ds(b, GCH)], i0_v)
        pltpu.sync_copy(src_hbm.at[pl.ds(b + GCH, GCH)], i1_v)
        c0 = pltpu.async_copy(hid_hbm.at[i0_v], b0_v, sem0)
        c1 = pltpu.async_copy(hid_hbm.at[i1_v], b1_v, sem1)
        c0.wait()
        c1.wait()
        pltpu.sync_copy(b0_v, out_hbm.at[pl.ds(b, GCH)])
        pltpu.sync_copy(b1_v, out_hbm.at[pl.ds(b + GCH, GCH)])
        return carry
    lax.fori_loop(0, (R // NW) // (2 * GCH), body, 0)


def _gemm_body(be_ref, xs_ref, ew_ref, eb_ref, ws_ref, out_ref):
    acc = lax.dot_general(xs_ref[...], ew_ref[0], (((1,), (1,)), ((), ())),
                          preferred_element_type=jnp.float32)
    out_ref[...] = (acc + eb_ref[0]) * ws_ref[...]


def _grouped_gemm(x_sorted, expert_w, expert_b, w_sorted, blk_e):
    grid_spec = pltpu.PrefetchScalarGridSpec(
        num_scalar_prefetch=1,
        grid=(NBLK,),
        in_specs=[
            pl.BlockSpec((BM, H), lambda i, be: (i, 0)),
            pl.BlockSpec((1, H, H), lambda i, be: (be[i], 0, 0)),
            pl.BlockSpec((1, 1, H), lambda i, be: (be[i], 0, 0)),
            pl.BlockSpec((BM, 1), lambda i, be: (i, 0)),
        ],
        out_specs=pl.BlockSpec((BM, H), lambda i, be: (i, 0)),
    )
    return pl.pallas_call(
        _gemm_body,
        grid_spec=grid_spec,
        out_shape=jax.ShapeDtypeStruct((R, H), jnp.float32),
    )(blk_e, x_sorted, expert_w, expert_b.reshape(E, 1, H),
      w_sorted.reshape(R, 1))


@functools.partial(
    pl.kernel,
    out_type=(jax.ShapeDtypeStruct((T, H), jnp.float32),
              jax.ShapeDtypeStruct((T, H), jnp.float32)),
    mesh=plsc.VectorSubcoreMesh(core_axis_name="c", subcore_axis_name="s"),
    scratch_types=[
        pltpu.VMEM((GCH,), jnp.int32),
        pltpu.VMEM((GCH,), jnp.int32),
        pltpu.VMEM((GCH, H), jnp.float32),
        pltpu.VMEM((GCH, H), jnp.float32),
        pltpu.SemaphoreType.DMA,
        pltpu.SemaphoreType.DMA,
    ],
)
def _sc_gather_pairs(outs_hbm, p0_hbm, p1_hbm, g0_hbm, g1_hbm,
                     i0_v, i1_v, b0_v, b1_v, sem0, sem1):
    base = _sc_worker_id() * (T // NW)
    def body(i, carry):
        b = base + i * GCH
        pltpu.sync_copy(p0_hbm.at[pl.ds(b, GCH)], i0_v)
        pltpu.sync_copy(p1_hbm.at[pl.ds(b, GCH)], i1_v)
        c0 = pltpu.async_copy(outs_hbm.at[i0_v], b0_v, sem0)
        c1 = pltpu.async_copy(outs_hbm.at[i1_v], b1_v, sem1)
        c0.wait()
        c1.wait()
        pltpu.sync_copy(b0_v, g0_hbm.at[pl.ds(b, GCH)])
        pltpu.sync_copy(b1_v, g1_hbm.at[pl.ds(b, GCH)])
        return carry
    lax.fori_loop(0, (T // NW) // GCH, body, 0)


def _add_body(a_ref, b_ref, o_ref):
    o_ref[...] = a_ref[...] + b_ref[...]


def _combine_add(g0, g1):
    return pl.pallas_call(
        _add_body,
        grid=(T // BT,),
        in_specs=[pl.BlockSpec((BT, H), lambda i: (i, 0)),
                  pl.BlockSpec((BT, H), lambda i: (i, 0))],
        out_specs=pl.BlockSpec((BT, H), lambda i: (i, 0)),
        out_shape=jax.ShapeDtypeStruct((T, H), jnp.float32),
    )(g0, g1)


def kernel(x, gate_w, gate_b, expert_w, expert_b):
    hidden = x.reshape(T, H)
    logits, top_idx, top_w = _router(hidden, gate_w, gate_b)

    flat_e = top_idx.reshape(-1)
    flat_w = top_w.reshape(-1)
    perm = jnp.argsort(flat_e, stable=True).astype(jnp.int32)
    sorted_e = flat_e[perm]
    counts = jnp.zeros((E,), jnp.int32).at[flat_e].add(1)
    padded = ((counts + BM - 1) // BM) * BM
    cpad = jnp.cumsum(padded)
    poff = cpad - padded
    coff = jnp.cumsum(counts) - counts
    j = jnp.arange(T * K, dtype=jnp.int32)
    dest = (poff[sorted_e] + j - coff[sorted_e]).astype(jnp.int32)
    src_row = jnp.zeros((R,), jnp.int32).at[dest].set(perm // K)
    w_sorted = jnp.zeros((R,), jnp.float32).at[dest].set(flat_w[perm])
    pos = jnp.zeros((T * K,), jnp.int32).at[perm].set(dest)
    p0 = pos[0::2]
    p1 = pos[1::2]
    blk_start = jnp.arange(NBLK, dtype=jnp.int32) * BM
    blk_e = jnp.minimum(
        jnp.searchsorted(cpad, blk_start, side="right").astype(jnp.int32),
        E - 1)

    x_sorted = _sc_gather_rows(hidden, src_row)
    out_sorted = _grouped_gemm(x_sorted, expert_w, expert_b, w_sorted, blk_e)
    g0, g1 = _sc_gather_pairs(out_sorted, p0, p1)
    final = _combine_add(g0, g1)
    return final.reshape(B, S, H), logits


# speedup vs baseline: 1.1713x; 1.1237x over previous
import functools

import jax
import jax.numpy as jnp
from jax import lax
from jax.experimental import pallas as pl
from jax.experimental.pallas import tpu as pltpu
from jax.experimental.pallas import tpu_sc as plsc

B, S, H, E, K = 4, 4096, 768, 64, 2
T = B * S
BM = 128
R = T * K + E * BM
NBLK = R // BM
BT = 1024

NC, NS = 2, 16
NW = NC * NS
GCH = 64


def _router_body(x_ref, gwt_ref, gb_ref, logits_ref, idx_ref, w_ref):
    xb = x_ref[...]
    logits = lax.dot_general(xb, gwt_ref[...], (((1,), (0,)), ((), ())),
                             preferred_element_type=jnp.float32)
    logits = logits + gb_ref[...]
    logits_ref[...] = logits
    ids = lax.broadcasted_iota(jnp.int32, logits.shape, 1)
    m1 = jnp.max(logits, axis=1, keepdims=True)
    i1 = jnp.min(jnp.where(logits == m1, ids, E), axis=1, keepdims=True)
    masked = jnp.where(ids == i1, -jnp.inf, logits)
    m2 = jnp.max(masked, axis=1, keepdims=True)
    i2 = jnp.min(jnp.where(masked == m2, ids, E), axis=1, keepdims=True)
    w1 = 1.0 / (1.0 + jnp.exp(m2 - m1))
    idx_ref[...] = jnp.concatenate([i1, i2], axis=1)
    w_ref[...] = jnp.concatenate([w1, 1.0 - w1], axis=1)


def _router(hidden, gate_w, gate_b):
    return pl.pallas_call(
        _router_body,
        grid=(T // BT,),
        in_specs=[
            pl.BlockSpec((BT, H), lambda i: (i, 0)),
            pl.BlockSpec((H, E), lambda i: (0, 0)),
            pl.BlockSpec((1, E), lambda i: (0, 0)),
        ],
        out_specs=[
            pl.BlockSpec((BT, E), lambda i: (i, 0)),
            pl.BlockSpec((BT, K), lambda i: (i, 0)),
            pl.BlockSpec((BT, K), lambda i: (i, 0)),
        ],
        out_shape=[
            jax.ShapeDtypeStruct((T, E), jnp.float32),
            jax.ShapeDtypeStruct((T, K), jnp.int32),
            jax.ShapeDtypeStruct((T, K), jnp.float32),
        ],
    )(hidden, gate_w.T, gate_b.reshape(1, E))


def _sc_worker_id():
    return lax.axis_index("s") * NC + lax.axis_index("c")


@functools.partial(
    pl.kernel,
    out_type=jax.ShapeDtypeStruct((R, H), jnp.float32),
    mesh=plsc.VectorSubcoreMesh(core_axis_name="c", subcore_axis_name="s"),
    scratch_types=[
        pltpu.VMEM((GCH,), jnp.int32),
        pltpu.VMEM((GCH,), jnp.int32),
        pltpu.VMEM((GCH, H), jnp.float32),
        pltpu.VMEM((GCH, H), jnp.float32),
        pltpu.SemaphoreType.DMA,
        pltpu.SemaphoreType.DMA,
    ],
)
def _sc_gather_rows(hid_hbm, src_hbm, out_hbm, i0_v, i1_v, b0_v, b1_v,
                    sem0, sem1):
    base = _sc_worker_id() * (R // NW)
    def body(i, carry):
        b = base + i * (2 * GCH)
        pltpu.sync_copy(src_hbm.at[pl.ds(b, GCH)], i0_v)
        pltpu.sync_copy(src_hbm.at[pl.ds(b + GCH, GCH)], i1_v)
        c0 = pltpu.async_copy(hid_hbm.at[i0_v], b0_v, sem0)
        c1 = pltpu.async_copy(hid_hbm.at[i1_v], b1_v, sem1)
        c0.wait()
        c1.wait()
        pltpu.sync_copy(b0_v, out_hbm.at[pl.ds(b, GCH)])
        pltpu.sync_copy(b1_v, out_hbm.at[pl.ds(b + GCH, GCH)])
        return carry
    lax.fori_loop(0, (R // NW) // (2 * GCH), body, 0)


@functools.partial(
    pl.kernel,
    out_type=jax.ShapeDtypeStruct((T, H), jnp.float32),
    mesh=plsc.VectorSubcoreMesh(core_axis_name="c", subcore_axis_name="s"),
    scratch_types=[
        pltpu.VMEM((GCH, H), jnp.float32),
        pltpu.VMEM((GCH, H), jnp.float32),
        pltpu.SemaphoreType.DMA,
        pltpu.SemaphoreType.DMA,
    ],
)
def _sc_linearize(hid_hbm, out_hbm, b0_v, b1_v, sem0, sem1):
    base = _sc_worker_id() * (T // NW)
    def body(i, carry):
        b = base + i * (2 * GCH)
        c0 = pltpu.async_copy(hid_hbm.at[pl.ds(b, GCH)], b0_v, sem0)
        c1 = pltpu.async_copy(hid_hbm.at[pl.ds(b + GCH, GCH)], b1_v, sem1)
        c0.wait()
        c1.wait()
        pltpu.sync_copy(b0_v, out_hbm.at[pl.ds(b, GCH)])
        pltpu.sync_copy(b1_v, out_hbm.at[pl.ds(b + GCH, GCH)])
        return carry
    lax.fori_loop(0, (T // NW) // (2 * GCH), body, 0)


def _gemm_body(be_ref, xs_ref, ew_ref, eb_ref, ws_ref, out_ref):
    acc = lax.dot_general(xs_ref[...], ew_ref[0], (((1,), (1,)), ((), ())),
                          preferred_element_type=jnp.float32)
    out_ref[...] = (acc + eb_ref[0]) * ws_ref[...]


def _grouped_gemm(x_sorted, expert_w, expert_b, w_sorted, blk_e):
    grid_spec = pltpu.PrefetchScalarGridSpec(
        num_scalar_prefetch=1,
        grid=(NBLK,),
        in_specs=[
            pl.BlockSpec((BM, H), lambda i, be: (i, 0)),
            pl.BlockSpec((1, H, H), lambda i, be: (be[i], 0, 0)),
            pl.BlockSpec((1, 1, H), lambda i, be: (be[i], 0, 0)),
            pl.BlockSpec((BM, 1), lambda i, be: (i, 0)),
        ],
        out_specs=pl.BlockSpec((BM, H), lambda i, be: (i, 0)),
    )
    return pl.pallas_call(
        _gemm_body,
        grid_spec=grid_spec,
        out_shape=jax.ShapeDtypeStruct((R, H), jnp.float32),
    )(blk_e, x_sorted, expert_w, expert_b.reshape(E, 1, H),
      w_sorted.reshape(R, 1))


@functools.partial(
    pl.kernel,
    out_type=(jax.ShapeDtypeStruct((T, H), jnp.float32),
              jax.ShapeDtypeStruct((T, H), jnp.float32)),
    mesh=plsc.VectorSubcoreMesh(core_axis_name="c", subcore_axis_name="s"),
    scratch_types=[
        pltpu.VMEM((GCH,), jnp.int32),
        pltpu.VMEM((GCH,), jnp.int32),
        pltpu.VMEM((GCH, H), jnp.float32),
        pltpu.VMEM((GCH, H), jnp.float32),
        pltpu.SemaphoreType.DMA,
        pltpu.SemaphoreType.DMA,
    ],
)
def _sc_gather_pairs(outs_hbm, p0_hbm, p1_hbm, g0_hbm, g1_hbm,
                     i0_v, i1_v, b0_v, b1_v, sem0, sem1):
    base = _sc_worker_id() * (T // NW)
    def body(i, carry):
        b = base + i * GCH
        pltpu.sync_copy(p0_hbm.at[pl.ds(b, GCH)], i0_v)
        pltpu.sync_copy(p1_hbm.at[pl.ds(b, GCH)], i1_v)
        c0 = pltpu.async_copy(outs_hbm.at[i0_v], b0_v, sem0)
        c1 = pltpu.async_copy(outs_hbm.at[i1_v], b1_v, sem1)
        c0.wait()
        c1.wait()
        pltpu.sync_copy(b0_v, g0_hbm.at[pl.ds(b, GCH)])
        pltpu.sync_copy(b1_v, g1_hbm.at[pl.ds(b, GCH)])
        return carry
    lax.fori_loop(0, (T // NW) // GCH, body, 0)


def _add_body(a_ref, b_ref, o_ref):
    o_ref[...] = a_ref[...] + b_ref[...]


def _combine_add(g0, g1):
    return pl.pallas_call(
        _add_body,
        grid=(T // BT,),
        in_specs=[pl.BlockSpec((BT, H), lambda i: (i, 0)),
                  pl.BlockSpec((BT, H), lambda i: (i, 0))],
        out_specs=pl.BlockSpec((BT, H), lambda i: (i, 0)),
        out_shape=jax.ShapeDtypeStruct((T, H), jnp.float32),
    )(g0, g1)


def kernel(x, gate_w, gate_b, expert_w, expert_b):
    hidden = x.reshape(T, H)
    logits, top_idx, top_w = _router(hidden, gate_w, gate_b)

    flat_e = top_idx.reshape(-1)
    flat_w = top_w.reshape(-1)
    perm = jnp.argsort(flat_e, stable=True).astype(jnp.int32)
    sorted_e = flat_e[perm]
    counts = jnp.zeros((E,), jnp.int32).at[flat_e].add(1)
    padded = ((counts + BM - 1) // BM) * BM
    cpad = jnp.cumsum(padded)
    poff = cpad - padded
    coff = jnp.cumsum(counts) - counts
    j = jnp.arange(T * K, dtype=jnp.int32)
    dest = (poff[sorted_e] + j - coff[sorted_e]).astype(jnp.int32)
    src_row = jnp.zeros((R,), jnp.int32).at[dest].set(perm // K)
    w_sorted = jnp.zeros((R,), jnp.float32).at[dest].set(flat_w[perm])
    pos = jnp.zeros((T * K,), jnp.int32).at[perm].set(dest)
    p0 = pos[0::2]
    p1 = pos[1::2]
    blk_start = jnp.arange(NBLK, dtype=jnp.int32) * BM
    blk_e = jnp.minimum(
        jnp.searchsorted(cpad, blk_start, side="right").astype(jnp.int32),
        E - 1)

    hidden_lin = _sc_linearize(hidden)
    x_sorted = _sc_gather_rows(hidden_lin, src_row)
    out_sorted = _grouped_gemm(x_sorted, expert_w, expert_b, w_sorted, blk_e)
    g0, g1 = _sc_gather_pairs(out_sorted, p0, p1)
    final = _combine_add(g0, g1)
    return final.reshape(B, S, H), logits


# speedup vs baseline: 1.2031x; 1.0272x over previous
import functools

import jax
import jax.numpy as jnp
from jax import lax
from jax.experimental import pallas as pl
from jax.experimental.pallas import tpu as pltpu
from jax.experimental.pallas import tpu_sc as plsc

B, S, H, E, K = 4, 4096, 768, 64, 2
T = B * S
BM = 128
R = T * K + E * BM
NBLK = R // BM
BT = 1024

NC, NS = 2, 16
NW = NC * NS
GCH = 64


def _router_body(x_ref, gwt_ref, gb_ref, logits_ref, idx_ref, w_ref):
    xb = x_ref[...]
    logits = lax.dot_general(xb, gwt_ref[...], (((1,), (0,)), ((), ())),
                             preferred_element_type=jnp.float32)
    logits = logits + gb_ref[...]
    logits_ref[...] = logits
    ids = lax.broadcasted_iota(jnp.int32, logits.shape, 1)
    m1 = jnp.max(logits, axis=1, keepdims=True)
    i1 = jnp.min(jnp.where(logits == m1, ids, E), axis=1, keepdims=True)
    masked = jnp.where(ids == i1, -jnp.inf, logits)
    m2 = jnp.max(masked, axis=1, keepdims=True)
    i2 = jnp.min(jnp.where(masked == m2, ids, E), axis=1, keepdims=True)
    w1 = 1.0 / (1.0 + jnp.exp(m2 - m1))
    idx_ref[...] = jnp.concatenate([i1, i2], axis=1)
    w_ref[...] = jnp.concatenate([w1, 1.0 - w1], axis=1)


def _router(hidden, gate_w, gate_b):
    return pl.pallas_call(
        _router_body,
        grid=(T // BT,),
        in_specs=[
            pl.BlockSpec((BT, H), lambda i: (i, 0)),
            pl.BlockSpec((H, E), lambda i: (0, 0)),
            pl.BlockSpec((1, E), lambda i: (0, 0)),
        ],
        out_specs=[
            pl.BlockSpec((BT, E), lambda i: (i, 0)),
            pl.BlockSpec((BT, K), lambda i: (i, 0)),
            pl.BlockSpec((BT, K), lambda i: (i, 0)),
        ],
        out_shape=[
            jax.ShapeDtypeStruct((T, E), jnp.float32),
            jax.ShapeDtypeStruct((T, K), jnp.int32),
            jax.ShapeDtypeStruct((T, K), jnp.float32),
        ],
    )(hidden, gate_w.T, gate_b.reshape(1, E))


def _sc_worker_id():
    return lax.axis_index("s") * NC + lax.axis_index("c")


@functools.partial(
    pl.kernel,
    out_type=jax.ShapeDtypeStruct((R, H), jnp.float32),
    mesh=plsc.VectorSubcoreMesh(core_axis_name="c", subcore_axis_name="s"),
    scratch_types=[
        pltpu.VMEM((GCH,), jnp.int32),
        pltpu.VMEM((GCH,), jnp.int32),
        pltpu.VMEM((GCH, H), jnp.float32),
        pltpu.VMEM((GCH, H), jnp.float32),
        pltpu.SemaphoreType.DMA,
        pltpu.SemaphoreType.DMA,
    ],
)
def _sc_gather_rows(hid_hbm, src_hbm, out_hbm, i0_v, i1_v, b0_v, b1_v,
                    sem0, sem1):
    base = _sc_worker_id() * (R // NW)
    def body(i, carry):
        b = base + i * (2 * GCH)
        pltpu.sync_copy(src_hbm.at[pl.ds(b, GCH)], i0_v)
        pltpu.sync_copy(src_hbm.at[pl.ds(b + GCH, GCH)], i1_v)
        c0 = pltpu.async_copy(hid_hbm.at[i0_v], b0_v, sem0)
        c1 = pltpu.async_copy(hid_hbm.at[i1_v], b1_v, sem1)
        c0.wait()
        c1.wait()
        pltpu.sync_copy(b0_v, out_hbm.at[pl.ds(b, GCH)])
        pltpu.sync_copy(b1_v, out_hbm.at[pl.ds(b + GCH, GCH)])
        return carry
    lax.fori_loop(0, (R // NW) // (2 * GCH), body, 0)


@functools.partial(
    pl.kernel,
    out_type=jax.ShapeDtypeStruct((T, H), jnp.float32),
    mesh=plsc.VectorSubcoreMesh(core_axis_name="c", subcore_axis_name="s"),
    scratch_types=[
        pltpu.VMEM((GCH, H), jnp.float32),
        pltpu.VMEM((GCH, H), jnp.float32),
        pltpu.SemaphoreType.DMA,
        pltpu.SemaphoreType.DMA,
    ],
)
def _sc_linearize(hid_hbm, out_hbm, b0_v, b1_v, sem0, sem1):
    base = _sc_worker_id() * (T // NW)
    def body(i, carry):
        b = base + i * (2 * GCH)
        c0 = pltpu.async_copy(hid_hbm.at[pl.ds(b, GCH)], b0_v, sem0)
        c1 = pltpu.async_copy(hid_hbm.at[pl.ds(b + GCH, GCH)], b1_v, sem1)
        c0.wait()
        c1.wait()
        pltpu.sync_copy(b0_v, out_hbm.at[pl.ds(b, GCH)])
        pltpu.sync_copy(b1_v, out_hbm.at[pl.ds(b + GCH, GCH)])
        return carry
    lax.fori_loop(0, (T // NW) // (2 * GCH), body, 0)


def _gemm_body(be_ref, xs_ref, ew_ref, eb_ref, ws_ref, out_ref):
    acc = lax.dot_general(xs_ref[...], ew_ref[0], (((1,), (1,)), ((), ())),
                          preferred_element_type=jnp.float32)
    out_ref[...] = (acc + eb_ref[0]) * ws_ref[...]


def _grouped_gemm(x_sorted, expert_w, expert_b, w_sorted, blk_e):
    grid_spec = pltpu.PrefetchScalarGridSpec(
        num_scalar_prefetch=1,
        grid=(NBLK,),
        in_specs=[
            pl.BlockSpec((BM, H), lambda i, be: (i, 0)),
            pl.BlockSpec((1, H, H), lambda i, be: (be[i], 0, 0)),
            pl.BlockSpec((1, 1, H), lambda i, be: (be[i], 0, 0)),
            pl.BlockSpec((BM, 1), lambda i, be: (i, 0)),
        ],
        out_specs=pl.BlockSpec((BM, H), lambda i, be: (i, 0)),
    )
    return pl.pallas_call(
        _gemm_body,
        grid_spec=grid_spec,
        out_shape=jax.ShapeDtypeStruct((R, H), jnp.float32),
    )(blk_e, x_sorted, expert_w, expert_b.reshape(E, 1, H),
      w_sorted.reshape(R, 1))


@functools.partial(
    pl.kernel,
    out_type=(jax.ShapeDtypeStruct((T, H), jnp.float32),
              jax.ShapeDtypeStruct((T, H), jnp.float32)),
    mesh=plsc.VectorSubcoreMesh(core_axis_name="c", subcore_axis_name="s"),
    scratch_types=[
        pltpu.VMEM((GCH,), jnp.int32),
        pltpu.VMEM((GCH,), jnp.int32),
        pltpu.VMEM((GCH, H), jnp.float32),
        pltpu.VMEM((GCH, H), jnp.float32),
        pltpu.SemaphoreType.DMA,
        pltpu.SemaphoreType.DMA,
    ],
)
def _sc_gather_pairs(outs_hbm, p0_hbm, p1_hbm, g0_hbm, g1_hbm,
                     i0_v, i1_v, b0_v, b1_v, sem0, sem1):
    base = _sc_worker_id() * (T // NW)
    def body(i, carry):
        b = base + i * GCH
        pltpu.sync_copy(p0_hbm.at[pl.ds(b, GCH)], i0_v)
        pltpu.sync_copy(p1_hbm.at[pl.ds(b, GCH)], i1_v)
        c0 = pltpu.async_copy(outs_hbm.at[i0_v], b0_v, sem0)
        c1 = pltpu.async_copy(outs_hbm.at[i1_v], b1_v, sem1)
        c0.wait()
        c1.wait()
        pltpu.sync_copy(b0_v, g0_hbm.at[pl.ds(b, GCH)])
        pltpu.sync_copy(b1_v, g1_hbm.at[pl.ds(b, GCH)])
        return carry
    lax.fori_loop(0, (T // NW) // GCH, body, 0)


def _add_body(a_ref, b_ref, o_ref):
    o_ref[...] = a_ref[...] + b_ref[...]


def _combine_add(g0, g1):
    return pl.pallas_call(
        _add_body,
        grid=(T // BT,),
        in_specs=[pl.BlockSpec((BT, H), lambda i: (i, 0)),
                  pl.BlockSpec((BT, H), lambda i: (i, 0))],
        out_specs=pl.BlockSpec((BT, H), lambda i: (i, 0)),
        out_shape=jax.ShapeDtypeStruct((T, H), jnp.float32),
    )(g0, g1)


def kernel(x, gate_w, gate_b, expert_w, expert_b):
    hidden = x.reshape(T, H)
    logits, top_idx, top_w = _router(hidden, gate_w, gate_b)

    flat_e = top_idx.reshape(-1)
    flat_w = top_w.reshape(-1)
    perm = jnp.argsort(flat_e, stable=True).astype(jnp.int32)
    sorted_e = flat_e[perm]
    counts = jnp.zeros((E,), jnp.int32).at[flat_e].add(1)
    padded = ((counts + BM - 1) // BM) * BM
    cpad = jnp.cumsum(padded)
    poff = cpad - padded
    coff = jnp.cumsum(counts) - counts
    j = jnp.arange(T * K, dtype=jnp.int32)
    dest = (poff[sorted_e] + j - coff[sorted_e]).astype(jnp.int32)
    src_row = (jnp.arange(R, dtype=jnp.int32) % T).at[dest].set(perm // K)
    w_sorted = jnp.zeros((R,), jnp.float32).at[dest].set(flat_w[perm])
    pos = jnp.zeros((T * K,), jnp.int32).at[perm].set(dest)
    p0 = pos[0::2]
    p1 = pos[1::2]
    blk_start = jnp.arange(NBLK, dtype=jnp.int32) * BM
    blk_e = jnp.minimum(
        jnp.searchsorted(cpad, blk_start, side="right").astype(jnp.int32),
        E - 1)

    hidden_lin = _sc_linearize(hidden)
    x_sorted = _sc_gather_rows(hidden_lin, src_row)
    out_sorted = _grouped_gemm(x_sorted, expert_w, expert_b, w_sorted, blk_e)
    g0, g1 = _sc_gather_pairs(out_sorted, p0, p1)
    final = _combine_add(g0, g1)
    return final.reshape(B, S, H), logits


# speedup vs baseline: 1.2994x; 1.0801x over previous
import functools

import jax
import jax.numpy as jnp
from jax import lax
from jax.experimental import pallas as pl
from jax.experimental.pallas import tpu as pltpu
from jax.experimental.pallas import tpu_sc as plsc

B, S, H, E, K = 4, 4096, 768, 64, 2
T = B * S
BM = 128
R = T * K + E * BM
NBLK = R // BM
BT = 1024

NC, NS = 2, 16
NW = NC * NS
GCH = 64


def _router_body(x_ref, gwt_ref, gb_ref, logits_ref, idx_ref, w_ref):
    xb = x_ref[...]
    logits = lax.dot_general(xb, gwt_ref[...], (((1,), (0,)), ((), ())),
                             preferred_element_type=jnp.float32)
    logits = logits + gb_ref[...]
    logits_ref[...] = logits
    ids = lax.broadcasted_iota(jnp.int32, logits.shape, 1)
    m1 = jnp.max(logits, axis=1, keepdims=True)
    i1 = jnp.min(jnp.where(logits == m1, ids, E), axis=1, keepdims=True)
    masked = jnp.where(ids == i1, -jnp.inf, logits)
    m2 = jnp.max(masked, axis=1, keepdims=True)
    i2 = jnp.min(jnp.where(masked == m2, ids, E), axis=1, keepdims=True)
    w1 = 1.0 / (1.0 + jnp.exp(m2 - m1))
    idx_ref[...] = jnp.concatenate([i1, i2], axis=1)
    w_ref[...] = jnp.concatenate([w1, 1.0 - w1], axis=1)


def _router(hidden, gate_w, gate_b):
    return pl.pallas_call(
        _router_body,
        grid=(T // BT,),
        in_specs=[
            pl.BlockSpec((BT, H), lambda i: (i, 0)),
            pl.BlockSpec((H, E), lambda i: (0, 0)),
            pl.BlockSpec((1, E), lambda i: (0, 0)),
        ],
        out_specs=[
            pl.BlockSpec((BT, E), lambda i: (i, 0)),
            pl.BlockSpec((BT, K), lambda i: (i, 0)),
            pl.BlockSpec((BT, K), lambda i: (i, 0)),
        ],
        out_shape=[
            jax.ShapeDtypeStruct((T, E), jnp.float32),
            jax.ShapeDtypeStruct((T, K), jnp.int32),
            jax.ShapeDtypeStruct((T, K), jnp.float32),
        ],
    )(hidden, gate_w.T, gate_b.reshape(1, E))


RCH = 2048
NRCH = (T * K) // RCH


def _rank_body(ids_ref, rank_ref, carry_ref):
    @pl.when(pl.program_id(0) == 0)
    def _():
        carry_ref[...] = jnp.zeros_like(carry_ref)
    ids = ids_ref[...]
    cols = lax.broadcasted_iota(jnp.int32, (RCH, E), 1)
    oh = (ids == cols).astype(jnp.int32)
    c = oh
    d = 1
    while d < RCH:
        c = c + jnp.concatenate(
            [jnp.zeros((d, E), jnp.int32), c[:RCH - d]], axis=0)
        d *= 2
    rank = jnp.sum((c - 1 + carry_ref[...]) * oh, axis=1, keepdims=True)
    rank_ref[...] = rank
    carry_ref[...] = carry_ref[...] + c[RCH - 1:RCH, :]


def _pair_ranks(flat_e):
    return pl.pallas_call(
        _rank_body,
        grid=(NRCH,),
        in_specs=[pl.BlockSpec((RCH, 1), lambda i: (i, 0))],
        out_specs=pl.BlockSpec((RCH, 1), lambda i: (i, 0)),
        out_shape=jax.ShapeDtypeStruct((T * K, 1), jnp.int32),
        scratch_shapes=[pltpu.VMEM((1, E), jnp.int32)],
        compiler_params=pltpu.CompilerParams(
            dimension_semantics=("arbitrary",)),
    )(flat_e.reshape(T * K, 1)).reshape(T * K)


def _sc_worker_id():
    return lax.axis_index("s") * NC + lax.axis_index("c")


@functools.partial(
    pl.kernel,
    out_type=jax.ShapeDtypeStruct((R, H), jnp.float32),
    mesh=plsc.VectorSubcoreMesh(core_axis_name="c", subcore_axis_name="s"),
    scratch_types=[
        pltpu.VMEM((GCH,), jnp.int32),
        pltpu.VMEM((GCH,), jnp.int32),
        pltpu.VMEM((GCH, H), jnp.float32),
        pltpu.VMEM((GCH, H), jnp.float32),
        pltpu.SemaphoreType.DMA,
        pltpu.SemaphoreType.DMA,
    ],
)
def _sc_gather_rows(hid_hbm, src_hbm, out_hbm, i0_v, i1_v, b0_v, b1_v,
                    sem0, sem1):
    base = _sc_worker_id() * (R // NW)
    def body(i, carry):
        b = base + i * (2 * GCH)
        pltpu.sync_copy(src_hbm.at[pl.ds(b, GCH)], i0_v)
        pltpu.sync_copy(src_hbm.at[pl.ds(b + GCH, GCH)], i1_v)
        c0 = pltpu.async_copy(hid_hbm.at[i0_v], b0_v, sem0)
        c1 = pltpu.async_copy(hid_hbm.at[i1_v], b1_v, sem1)
        c0.wait()
        c1.wait()
        pltpu.sync_copy(b0_v, out_hbm.at[pl.ds(b, GCH)])
        pltpu.sync_copy(b1_v, out_hbm.at[pl.ds(b + GCH, GCH)])
        return carry
    lax.fori_loop(0, (R // NW) // (2 * GCH), body, 0)


@functools.partial(
    pl.kernel,
    out_type=jax.ShapeDtypeStruct((T, H), jnp.float32),
    mesh=plsc.VectorSubcoreMesh(core_axis_name="c", subcore_axis_name="s"),
    scratch_types=[
        pltpu.VMEM((GCH, H), jnp.float32),
        pltpu.VMEM((GCH, H), jnp.float32),
        pltpu.SemaphoreType.DMA,
        pltpu.SemaphoreType.DMA,
    ],
)
def _sc_linearize(hid_hbm, out_hbm, b0_v, b1_v, sem0, sem1):
    base = _sc_worker_id() * (T // NW)
    def body(i, carry):
        b = base + i * (2 * GCH)
        c0 = pltpu.async_copy(hid_hbm.at[pl.ds(b, GCH)], b0_v, sem0)
        c1 = pltpu.async_copy(hid_hbm.at[pl.ds(b + GCH, GCH)], b1_v, sem1)
        c0.wait()
        c1.wait()
        pltpu.sync_copy(b0_v, out_hbm.at[pl.ds(b, GCH)])
        pltpu.sync_copy(b1_v, out_hbm.at[pl.ds(b + GCH, GCH)])
        return carry
    lax.fori_loop(0, (T // NW) // (2 * GCH), body, 0)


def _gemm_body(be_ref, xs_ref, ew_ref, eb_ref, ws_ref, out_ref):
    acc = lax.dot_general(xs_ref[...], ew_ref[0], (((1,), (1,)), ((), ())),
                          preferred_element_type=jnp.float32)
    out_ref[...] = (acc + eb_ref[0]) * ws_ref[...]


def _grouped_gemm(x_sorted, expert_w, expert_b, w_sorted, blk_e):
    grid_spec = pltpu.PrefetchScalarGridSpec(
        num_scalar_prefetch=1,
        grid=(NBLK,),
        in_specs=[
            pl.BlockSpec((BM, H), lambda i, be: (i, 0)),
            pl.BlockSpec((1, H, H), lambda i, be: (be[i], 0, 0)),
            pl.BlockSpec((1, 1, H), lambda i, be: (be[i], 0, 0)),
            pl.BlockSpec((BM, 1), lambda i, be: (i, 0)),
        ],
        out_specs=pl.BlockSpec((BM, H), lambda i, be: (i, 0)),
    )
    return pl.pallas_call(
        _gemm_body,
        grid_spec=grid_spec,
        out_shape=jax.ShapeDtypeStruct((R, H), jnp.float32),
    )(blk_e, x_sorted, expert_w, expert_b.reshape(E, 1, H),
      w_sorted.reshape(R, 1))


@functools.partial(
    pl.kernel,
    out_type=(jax.ShapeDtypeStruct((T, H), jnp.float32),
              jax.ShapeDtypeStruct((T, H), jnp.float32)),
    mesh=plsc.VectorSubcoreMesh(core_axis_name="c", subcore_axis_name="s"),
    scratch_types=[
        pltpu.VMEM((GCH,), jnp.int32),
        pltpu.VMEM((GCH,), jnp.int32),
        pltpu.VMEM((GCH, H), jnp.float32),
        pltpu.VMEM((GCH, H), jnp.float32),
        pltpu.SemaphoreType.DMA,
        pltpu.SemaphoreType.DMA,
    ],
)
def _sc_gather_pairs(outs_hbm, p0_hbm, p1_hbm, g0_hbm, g1_hbm,
                     i0_v, i1_v, b0_v, b1_v, sem0, sem1):
    base = _sc_worker_id() * (T // NW)
    def body(i, carry):
        b = base + i * GCH
        pltpu.sync_copy(p0_hbm.at[pl.ds(b, GCH)], i0_v)
        pltpu.sync_copy(p1_hbm.at[pl.ds(b, GCH)], i1_v)
        c0 = pltpu.async_copy(outs_hbm.at[i0_v], b0_v, sem0)
        c1 = pltpu.async_copy(outs_hbm.at[i1_v], b1_v, sem1)
        c0.wait()
        c1.wait()
        pltpu.sync_copy(b0_v, g0_hbm.at[pl.ds(b, GCH)])
        pltpu.sync_copy(b1_v, g1_hbm.at[pl.ds(b, GCH)])
        return carry
    lax.fori_loop(0, (T // NW) // GCH, body, 0)


def _add_body(a_ref, b_ref, o_ref):
    o_ref[...] = a_ref[...] + b_ref[...]


def _combine_add(g0, g1):
    return pl.pallas_call(
        _add_body,
        grid=(T // BT,),
        in_specs=[pl.BlockSpec((BT, H), lambda i: (i, 0)),
                  pl.BlockSpec((BT, H), lambda i: (i, 0))],
        out_specs=pl.BlockSpec((BT, H), lambda i: (i, 0)),
        out_shape=jax.ShapeDtypeStruct((T, H), jnp.float32),
    )(g0, g1)


def kernel(x, gate_w, gate_b, expert_w, expert_b):
    hidden = x.reshape(T, H)
    logits, top_idx, top_w = _router(hidden, gate_w, gate_b)

    flat_e = top_idx.reshape(-1)
    flat_w = top_w.reshape(-1)
    rank = _pair_ranks(flat_e)
    counts = jnp.zeros((E,), jnp.int32).at[flat_e].add(1)
    padded = ((counts + BM - 1) // BM) * BM
    cpad = jnp.cumsum(padded)
    poff = cpad - padded
    dest = (poff[flat_e] + rank).astype(jnp.int32)
    j = jnp.arange(T * K, dtype=jnp.int32)
    src_row = (jnp.arange(R, dtype=jnp.int32) % T).at[dest].set(j // K)
    w_sorted = jnp.zeros((R,), jnp.float32).at[dest].set(flat_w)
    p0 = dest[0::2]
    p1 = dest[1::2]
    blk_start = jnp.arange(NBLK, dtype=jnp.int32) * BM
    blk_e = jnp.minimum(
        jnp.searchsorted(cpad, blk_start, side="right").astype(jnp.int32),
        E - 1)

    hidden_lin = _sc_linearize(hidden)
    x_sorted = _sc_gather_rows(hidden_lin, src_row)
    out_sorted = _grouped_gemm(x_sorted, expert_w, expert_b, w_sorted, blk_e)
    g0, g1 = _sc_gather_pairs(out_sorted, p0, p1)
    final = _combine_add(g0, g1)
    return final.reshape(B, S, H), logits


# speedup vs baseline: 1.6627x; 1.2796x over previous
import functools

import jax
import jax.numpy as jnp
from jax import lax
from jax.experimental import pallas as pl
from jax.experimental.pallas import tpu as pltpu
from jax.experimental.pallas import tpu_sc as plsc

B, S, H, E, K = 4, 4096, 768, 64, 2
T = B * S
BM = 128
R = T * K + E * BM
NBLK = R // BM
BT = 1024

NC, NS = 2, 16
NW = NC * NS
GCH = 64


def _router_body(x_ref, gwt_ref, gb_ref, logits_ref, idx_ref, w_ref):
    xb = x_ref[...]
    logits = lax.dot_general(xb, gwt_ref[...], (((1,), (0,)), ((), ())),
                             preferred_element_type=jnp.float32)
    logits = logits + gb_ref[...]
    logits_ref[...] = logits
    ids = lax.broadcasted_iota(jnp.int32, logits.shape, 1)
    m1 = jnp.max(logits, axis=1, keepdims=True)
    i1 = jnp.min(jnp.where(logits == m1, ids, E), axis=1, keepdims=True)
    masked = jnp.where(ids == i1, -jnp.inf, logits)
    m2 = jnp.max(masked, axis=1, keepdims=True)
    i2 = jnp.min(jnp.where(masked == m2, ids, E), axis=1, keepdims=True)
    w1 = 1.0 / (1.0 + jnp.exp(m2 - m1))
    idx_ref[...] = jnp.concatenate([i1, i2], axis=1)
    w_ref[...] = jnp.concatenate([w1, 1.0 - w1], axis=1)


def _router(hidden, gate_w, gate_b):
    return pl.pallas_call(
        _router_body,
        grid=(T // BT,),
        in_specs=[
            pl.BlockSpec((BT, H), lambda i: (i, 0)),
            pl.BlockSpec((H, E), lambda i: (0, 0)),
            pl.BlockSpec((1, E), lambda i: (0, 0)),
        ],
        out_specs=[
            pl.BlockSpec((BT, E), lambda i: (i, 0)),
            pl.BlockSpec((BT, K), lambda i: (i, 0)),
            pl.BlockSpec((BT, K), lambda i: (i, 0)),
        ],
        out_shape=[
            jax.ShapeDtypeStruct((T, E), jnp.float32),
            jax.ShapeDtypeStruct((T, K), jnp.int32),
            jax.ShapeDtypeStruct((T, K), jnp.float32),
        ],
    )(hidden, gate_w.T, gate_b.reshape(1, E))


RCH = 2048
NRCH = (T * K) // RCH


def _rank_body(ids_ref, rank_ref, carry_ref):
    @pl.when(pl.program_id(0) == 0)
    def _():
        carry_ref[...] = jnp.zeros_like(carry_ref)
    ids = ids_ref[...]
    cols = lax.broadcasted_iota(jnp.int32, (RCH, E), 1)
    oh = (ids == cols).astype(jnp.int32)
    c = oh
    d = 1
    while d < RCH:
        c = c + jnp.concatenate(
            [jnp.zeros((d, E), jnp.int32), c[:RCH - d]], axis=0)
        d *= 2
    rank = jnp.sum((c - 1 + carry_ref[...]) * oh, axis=1, keepdims=True)
    rank_ref[...] = rank
    carry_ref[...] = carry_ref[...] + c[RCH - 1:RCH, :]


def _pair_ranks(flat_e):
    return pl.pallas_call(
        _rank_body,
        grid=(NRCH,),
        in_specs=[pl.BlockSpec((RCH, 1), lambda i: (i, 0))],
        out_specs=pl.BlockSpec((RCH, 1), lambda i: (i, 0)),
        out_shape=jax.ShapeDtypeStruct((T * K, 1), jnp.int32),
        scratch_shapes=[pltpu.VMEM((1, E), jnp.int32)],
        compiler_params=pltpu.CompilerParams(
            dimension_semantics=("arbitrary",)),
    )(flat_e.reshape(T * K, 1)).reshape(T * K)


def _sc_worker_id():
    return lax.axis_index("s") * NC + lax.axis_index("c")


@functools.partial(
    pl.kernel,
    out_type=(jax.ShapeDtypeStruct((R, H), jnp.float32),
              jax.ShapeDtypeStruct((R,), jnp.float32)),
    mesh=plsc.VectorSubcoreMesh(core_axis_name="c", subcore_axis_name="s"),
    scratch_types=[
        pltpu.VMEM((GCH,), jnp.int32),
        pltpu.VMEM((GCH,), jnp.int32),
        pltpu.VMEM((GCH,), jnp.float32),
        pltpu.VMEM((GCH,), jnp.float32),
        pltpu.VMEM((GCH, H), jnp.float32),
        pltpu.SemaphoreType.DMA,
        pltpu.SemaphoreType.DMA,
        pltpu.SemaphoreType.DMA,
        pltpu.SemaphoreType.DMA,
    ],
)
def _sc_dispatch(hid_hbm, d0_hbm, d1_hbm, w0_hbm, w1_hbm,
                 xs_hbm, ws_hbm, i0_v, i1_v, w0_v, w1_v, buf_v,
                 sem0, sem1, sem2, sem3):
    base = _sc_worker_id() * (T // NW)
    def body(i, carry):
        b = base + i * GCH
        pltpu.sync_copy(d0_hbm.at[pl.ds(b, GCH)], i0_v)
        pltpu.sync_copy(d1_hbm.at[pl.ds(b, GCH)], i1_v)
        pltpu.sync_copy(w0_hbm.at[pl.ds(b, GCH)], w0_v)
        pltpu.sync_copy(w1_hbm.at[pl.ds(b, GCH)], w1_v)
        pltpu.sync_copy(hid_hbm.at[pl.ds(b, GCH)], buf_v)
        c0 = pltpu.async_copy(buf_v, xs_hbm.at[i0_v], sem0)
        c1 = pltpu.async_copy(buf_v, xs_hbm.at[i1_v], sem1)
        c2 = pltpu.async_copy(w0_v, ws_hbm.at[i0_v], sem2)
        c3 = pltpu.async_copy(w1_v, ws_hbm.at[i1_v], sem3)
        c0.wait()
        c1.wait()
        c2.wait()
        c3.wait()
        return carry
    lax.fori_loop(0, (T // NW) // GCH, body, 0)


def _gemm_body(be_ref, xs_ref, ew_ref, eb_ref, ws_ref, out_ref):
    acc = lax.dot_general(xs_ref[...], ew_ref[0], (((1,), (1,)), ((), ())),
                          preferred_element_type=jnp.float32)
    out_ref[...] = (acc + eb_ref[0]) * ws_ref[...]


def _grouped_gemm(x_sorted, expert_w, expert_b, w_sorted, blk_e):
    grid_spec = pltpu.PrefetchScalarGridSpec(
        num_scalar_prefetch=1,
        grid=(NBLK,),
        in_specs=[
            pl.BlockSpec((BM, H), lambda i, be: (i, 0)),
            pl.BlockSpec((1, H, H), lambda i, be: (be[i], 0, 0)),
            pl.BlockSpec((1, 1, H), lambda i, be: (be[i], 0, 0)),
            pl.BlockSpec((BM, 1), lambda i, be: (i, 0)),
        ],
        out_specs=pl.BlockSpec((BM, H), lambda i, be: (i, 0)),
    )
    return pl.pallas_call(
        _gemm_body,
        grid_spec=grid_spec,
        out_shape=jax.ShapeDtypeStruct((R, H), jnp.float32),
    )(blk_e, x_sorted, expert_w, expert_b.reshape(E, 1, H),
      w_sorted.reshape(R, 1))


@functools.partial(
    pl.kernel,
    out_type=(jax.ShapeDtypeStruct((T, H), jnp.float32),
              jax.ShapeDtypeStruct((T, H), jnp.float32)),
    mesh=plsc.VectorSubcoreMesh(core_axis_name="c", subcore_axis_name="s"),
    scratch_types=[
        pltpu.VMEM((GCH,), jnp.int32),
        pltpu.VMEM((GCH,), jnp.int32),
        pltpu.VMEM((GCH, H), jnp.float32),
        pltpu.VMEM((GCH, H), jnp.float32),
        pltpu.SemaphoreType.DMA,
        pltpu.SemaphoreType.DMA,
    ],
)
def _sc_gather_pairs(outs_hbm, p0_hbm, p1_hbm, g0_hbm, g1_hbm,
                     i0_v, i1_v, b0_v, b1_v, sem0, sem1):
    base = _sc_worker_id() * (T // NW)
    def body(i, carry):
        b = base + i * GCH
        pltpu.sync_copy(p0_hbm.at[pl.ds(b, GCH)], i0_v)
        pltpu.sync_copy(p1_hbm.at[pl.ds(b, GCH)], i1_v)
        c0 = pltpu.async_copy(outs_hbm.at[i0_v], b0_v, sem0)
        c1 = pltpu.async_copy(outs_hbm.at[i1_v], b1_v, sem1)
        c0.wait()
        c1.wait()
        pltpu.sync_copy(b0_v, g0_hbm.at[pl.ds(b, GCH)])
        pltpu.sync_copy(b1_v, g1_hbm.at[pl.ds(b, GCH)])
        return carry
    lax.fori_loop(0, (T // NW) // GCH, body, 0)


def _add_body(a_ref, b_ref, o_ref):
    o_ref[...] = a_ref[...] + b_ref[...]


def _combine_add(g0, g1):
    return pl.pallas_call(
        _add_body,
        grid=(T // BT,),
        in_specs=[pl.BlockSpec((BT, H), lambda i: (i, 0)),
                  pl.BlockSpec((BT, H), lambda i: (i, 0))],
        out_specs=pl.BlockSpec((BT, H), lambda i: (i, 0)),
        out_shape=jax.ShapeDtypeStruct((T, H), jnp.float32),
    )(g0, g1)


def kernel(x, gate_w, gate_b, expert_w, expert_b):
    hidden = x.reshape(T, H)
    logits, top_idx, top_w = _router(hidden, gate_w, gate_b)

    flat_e = top_idx.reshape(-1)
    flat_w = top_w.reshape(-1)
    rank = _pair_ranks(flat_e)
    counts = jnp.zeros((E,), jnp.int32).at[flat_e].add(1)
    padded = ((counts + BM - 1) // BM) * BM
    cpad = jnp.cumsum(padded)
    poff = cpad - padded
    dest = (poff[flat_e] + rank).astype(jnp.int32)
    d2 = dest.reshape(T, K)
    p0 = d2[:, 0]
    p1 = d2[:, 1]
    w0 = top_w[:, 0]
    w1 = top_w[:, 1]
    blk_start = jnp.arange(NBLK, dtype=jnp.int32) * BM
    blk_e = jnp.minimum(
        jnp.searchsorted(cpad, blk_start, side="right").astype(jnp.int32),
        E - 1)

    x_sorted, w_sorted = _sc_dispatch(hidden, p0, p1, w0, w1)
    out_sorted = _grouped_gemm(x_sorted, expert_w, expert_b, w_sorted, blk_e)
    g0, g1 = _sc_gather_pairs(out_sorted, p0, p1)
    final = _combine_add(g0, g1)
    return final.reshape(B, S, H), logits


# speedup vs baseline: 1.6632x; 1.0003x over previous
import functools

import jax
import jax.numpy as jnp
from jax import lax
from jax.experimental import pallas as pl
from jax.experimental.pallas import tpu as pltpu
from jax.experimental.pallas import tpu_sc as plsc

B, S, H, E, K = 4, 4096, 768, 64, 2
T = B * S
BM = 128
R = T * K + E * BM
NBLK = R // BM
BT = 1024

NC, NS = 2, 16
NW = NC * NS
GCH = 64


def _router_body(x_ref, gwt_ref, gb_ref, logits_ref, idx_ref, w_ref):
    xb = x_ref[...]
    logits = lax.dot_general(xb, gwt_ref[...], (((1,), (0,)), ((), ())),
                             preferred_element_type=jnp.float32)
    logits = logits + gb_ref[...]
    logits_ref[...] = logits
    ids = lax.broadcasted_iota(jnp.int32, logits.shape, 1)
    m1 = jnp.max(logits, axis=1, keepdims=True)
    i1 = jnp.min(jnp.where(logits == m1, ids, E), axis=1, keepdims=True)
    masked = jnp.where(ids == i1, -jnp.inf, logits)
    m2 = jnp.max(masked, axis=1, keepdims=True)
    i2 = jnp.min(jnp.where(masked == m2, ids, E), axis=1, keepdims=True)
    w1 = 1.0 / (1.0 + jnp.exp(m2 - m1))
    idx_ref[...] = jnp.concatenate([i1, i2], axis=1)
    w_ref[...] = jnp.concatenate([w1, 1.0 - w1], axis=1)


def _router(hidden, gate_w, gate_b):
    return pl.pallas_call(
        _router_body,
        grid=(T // BT,),
        in_specs=[
            pl.BlockSpec((BT, H), lambda i: (i, 0)),
            pl.BlockSpec((H, E), lambda i: (0, 0)),
            pl.BlockSpec((1, E), lambda i: (0, 0)),
        ],
        out_specs=[
            pl.BlockSpec((BT, E), lambda i: (i, 0)),
            pl.BlockSpec((BT, K), lambda i: (i, 0)),
            pl.BlockSpec((BT, K), lambda i: (i, 0)),
        ],
        out_shape=[
            jax.ShapeDtypeStruct((T, E), jnp.float32),
            jax.ShapeDtypeStruct((T, K), jnp.int32),
            jax.ShapeDtypeStruct((T, K), jnp.float32),
        ],
    )(hidden, gate_w.T, gate_b.reshape(1, E))


RCH = 2048
NRCH = (T * K) // RCH


def _rank_body(ids_ref, rank_ref, carry_ref):
    @pl.when(pl.program_id(0) == 0)
    def _():
        carry_ref[...] = jnp.zeros_like(carry_ref)
    ids = ids_ref[...]
    cols = lax.broadcasted_iota(jnp.int32, (RCH, E), 1)
    oh = (ids == cols).astype(jnp.int32)
    c = oh
    d = 1
    while d < RCH:
        c = c + jnp.concatenate(
            [jnp.zeros((d, E), jnp.int32), c[:RCH - d]], axis=0)
        d *= 2
    rank = jnp.sum((c - 1 + carry_ref[...]) * oh, axis=1, keepdims=True)
    rank_ref[...] = rank
    carry_ref[...] = carry_ref[...] + c[RCH - 1:RCH, :]


def _pair_ranks(flat_e):
    return pl.pallas_call(
        _rank_body,
        grid=(NRCH,),
        in_specs=[pl.BlockSpec((RCH, 1), lambda i: (i, 0))],
        out_specs=pl.BlockSpec((RCH, 1), lambda i: (i, 0)),
        out_shape=jax.ShapeDtypeStruct((T * K, 1), jnp.int32),
        scratch_shapes=[pltpu.VMEM((1, E), jnp.int32)],
        compiler_params=pltpu.CompilerParams(
            dimension_semantics=("arbitrary",)),
    )(flat_e.reshape(T * K, 1)).reshape(T * K)


def _sc_worker_id():
    return lax.axis_index("s") * NC + lax.axis_index("c")


@functools.partial(
    pl.kernel,
    out_type=(jax.ShapeDtypeStruct((R, H), jnp.float32),
              jax.ShapeDtypeStruct((R,), jnp.float32)),
    mesh=plsc.VectorSubcoreMesh(core_axis_name="c", subcore_axis_name="s"),
    scratch_types=[
        pltpu.VMEM((GCH,), jnp.int32),
        pltpu.VMEM((GCH,), jnp.int32),
        pltpu.VMEM((GCH,), jnp.float32),
        pltpu.VMEM((GCH,), jnp.float32),
        pltpu.VMEM((GCH, H), jnp.float32),
        pltpu.SemaphoreType.DMA,
        pltpu.SemaphoreType.DMA,
        pltpu.SemaphoreType.DMA,
        pltpu.SemaphoreType.DMA,
    ],
)
def _sc_dispatch(hid_hbm, d0_hbm, d1_hbm, w0_hbm, w1_hbm,
                 xs_hbm, ws_hbm, i0_v, i1_v, w0_v, w1_v, buf_v,
                 sem0, sem1, sem2, sem3):
    base = _sc_worker_id() * (T // NW)
    def body(i, carry):
        b = base + i * GCH
        pltpu.sync_copy(d0_hbm.at[pl.ds(b, GCH)], i0_v)
        pltpu.sync_copy(d1_hbm.at[pl.ds(b, GCH)], i1_v)
        pltpu.sync_copy(w0_hbm.at[pl.ds(b, GCH)], w0_v)
        pltpu.sync_copy(w1_hbm.at[pl.ds(b, GCH)], w1_v)
        pltpu.sync_copy(hid_hbm.at[pl.ds(b, GCH)], buf_v)
        c0 = pltpu.async_copy(buf_v, xs_hbm.at[i0_v], sem0)
        c1 = pltpu.async_copy(buf_v, xs_hbm.at[i1_v], sem1)
        c2 = pltpu.async_copy(w0_v, ws_hbm.at[i0_v], sem2)
        c3 = pltpu.async_copy(w1_v, ws_hbm.at[i1_v], sem3)
        c0.wait()
        c1.wait()
        c2.wait()
        c3.wait()
        return carry
    lax.fori_loop(0, (T // NW) // GCH, body, 0)


def _gemm_body(be_ref, xs_ref, ew_ref, eb_ref, ws_ref, out_ref):
    acc = lax.dot_general(xs_ref[...].astype(jnp.bfloat16),
                          ew_ref[0].astype(jnp.bfloat16),
                          (((1,), (1,)), ((), ())),
                          preferred_element_type=jnp.float32)
    out_ref[...] = (acc + eb_ref[0]) * ws_ref[...]


def _grouped_gemm(x_sorted, expert_w, expert_b, w_sorted, blk_e):
    grid_spec = pltpu.PrefetchScalarGridSpec(
        num_scalar_prefetch=1,
        grid=(NBLK,),
        in_specs=[
            pl.BlockSpec((BM, H), lambda i, be: (i, 0)),
            pl.BlockSpec((1, H, H), lambda i, be: (be[i], 0, 0)),
            pl.BlockSpec((1, 1, H), lambda i, be: (be[i], 0, 0)),
            pl.BlockSpec((BM, 1), lambda i, be: (i, 0)),
        ],
        out_specs=pl.BlockSpec((BM, H), lambda i, be: (i, 0)),
    )
    return pl.pallas_call(
        _gemm_body,
        grid_spec=grid_spec,
        out_shape=jax.ShapeDtypeStruct((R, H), jnp.float32),
    )(blk_e, x_sorted, expert_w, expert_b.reshape(E, 1, H),
      w_sorted.reshape(R, 1))


@functools.partial(
    pl.kernel,
    out_type=(jax.ShapeDtypeStruct((T, H), jnp.float32),
              jax.ShapeDtypeStruct((T, H), jnp.float32)),
    mesh=plsc.VectorSubcoreMesh(core_axis_name="c", subcore_axis_name="s"),
    scratch_types=[
        pltpu.VMEM((GCH,), jnp.int32),
        pltpu.VMEM((GCH,), jnp.int32),
        pltpu.VMEM((GCH, H), jnp.float32),
        pltpu.VMEM((GCH, H), jnp.float32),
        pltpu.SemaphoreType.DMA,
        pltpu.SemaphoreType.DMA,
    ],
)
def _sc_gather_pairs(outs_hbm, p0_hbm, p1_hbm, g0_hbm, g1_hbm,
                     i0_v, i1_v, b0_v, b1_v, sem0, sem1):
    base = _sc_worker_id() * (T // NW)
    def body(i, carry):
        b = base + i * GCH
        pltpu.sync_copy(p0_hbm.at[pl.ds(b, GCH)], i0_v)
        pltpu.sync_copy(p1_hbm.at[pl.ds(b, GCH)], i1_v)
        c0 = pltpu.async_copy(outs_hbm.at[i0_v], b0_v, sem0)
        c1 = pltpu.async_copy(outs_hbm.at[i1_v], b1_v, sem1)
        c0.wait()
        c1.wait()
        pltpu.sync_copy(b0_v, g0_hbm.at[pl.ds(b, GCH)])
        pltpu.sync_copy(b1_v, g1_hbm.at[pl.ds(b, GCH)])
        return carry
    lax.fori_loop(0, (T // NW) // GCH, body, 0)


def _add_body(a_ref, b_ref, o_ref):
    o_ref[...] = a_ref[...] + b_ref[...]


def _combine_add(g0, g1):
    return pl.pallas_call(
        _add_body,
        grid=(T // BT,),
        in_specs=[pl.BlockSpec((BT, H), lambda i: (i, 0)),
                  pl.BlockSpec((BT, H), lambda i: (i, 0))],
        out_specs=pl.BlockSpec((BT, H), lambda i: (i, 0)),
        out_shape=jax.ShapeDtypeStruct((T, H), jnp.float32),
    )(g0, g1)


def kernel(x, gate_w, gate_b, expert_w, expert_b):
    hidden = x.reshape(T, H)
    logits, top_idx, top_w = _router(hidden, gate_w, gate_b)

    flat_e = top_idx.reshape(-1)
    flat_w = top_w.reshape(-1)
    rank = _pair_ranks(flat_e)
    counts = jnp.zeros((E,), jnp.int32).at[flat_e].add(1)
    padded = ((counts + BM - 1) // BM) * BM
    cpad = jnp.cumsum(padded)
    poff = cpad - padded
    dest = (poff[flat_e] + rank).astype(jnp.int32)
    d2 = dest.reshape(T, K)
    p0 = d2[:, 0]
    p1 = d2[:, 1]
    w0 = top_w[:, 0]
    w1 = top_w[:, 1]
    blk_start = jnp.arange(NBLK, dtype=jnp.int32) * BM
    blk_e = jnp.minimum(
        jnp.searchsorted(cpad, blk_start, side="right").astype(jnp.int32),
        E - 1)

    x_sorted, w_sorted = _sc_dispatch(hidden, p0, p1, w0, w1)
    out_sorted = _grouped_gemm(x_sorted, expert_w, expert_b, w_sorted, blk_e)
    g0, g1 = _sc_gather_pairs(out_sorted, p0, p1)
    final = _combine_add(g0, g1)
    return final.reshape(B, S, H), logits


# speedup vs baseline: 2.2848x; 1.3737x over previous
import functools

import jax
import jax.numpy as jnp
from jax import lax
from jax.experimental import pallas as pl
from jax.experimental.pallas import tpu as pltpu
from jax.experimental.pallas import tpu_sc as plsc

B, S, H, E, K = 4, 4096, 768, 64, 2
T = B * S
BM = 128
R = T * K + E * BM
NBLK = R // BM
BT = 1024

NC, NS = 2, 16
NW = NC * NS
GCH = 64


def _router_body(x_ref, gwt_ref, gb_ref, logits_ref, idx_ref, w_ref):
    xb = x_ref[...]
    logits = lax.dot_general(xb, gwt_ref[...], (((1,), (0,)), ((), ())),
                             preferred_element_type=jnp.float32)
    logits = logits + gb_ref[...]
    logits_ref[...] = logits
    ids = lax.broadcasted_iota(jnp.int32, logits.shape, 1)
    m1 = jnp.max(logits, axis=1, keepdims=True)
    i1 = jnp.min(jnp.where(logits == m1, ids, E), axis=1, keepdims=True)
    masked = jnp.where(ids == i1, -jnp.inf, logits)
    m2 = jnp.max(masked, axis=1, keepdims=True)
    i2 = jnp.min(jnp.where(masked == m2, ids, E), axis=1, keepdims=True)
    w1 = 1.0 / (1.0 + jnp.exp(m2 - m1))
    idx_ref[...] = jnp.concatenate([i1, i2], axis=1)
    w_ref[...] = jnp.concatenate([w1, 1.0 - w1], axis=1)


def _router(hidden, gate_w, gate_b):
    return pl.pallas_call(
        _router_body,
        grid=(T // BT,),
        in_specs=[
            pl.BlockSpec((BT, H), lambda i: (i, 0)),
            pl.BlockSpec((H, E), lambda i: (0, 0)),
            pl.BlockSpec((1, E), lambda i: (0, 0)),
        ],
        out_specs=[
            pl.BlockSpec((BT, E), lambda i: (i, 0)),
            pl.BlockSpec((BT, K), lambda i: (i, 0)),
            pl.BlockSpec((BT, K), lambda i: (i, 0)),
        ],
        out_shape=[
            jax.ShapeDtypeStruct((T, E), jnp.float32),
            jax.ShapeDtypeStruct((T, K), jnp.int32),
            jax.ShapeDtypeStruct((T, K), jnp.float32),
        ],
    )(hidden, gate_w.T, gate_b.reshape(1, E))


RCH = 2048
NRCH = (T * K) // RCH


def _rank_body(ids_ref, poff_ref, dest_ref, carry_ref):
    @pl.when(pl.program_id(0) == 0)
    def _():
        carry_ref[...] = jnp.zeros_like(carry_ref)
    ids = ids_ref[...]
    cols = lax.broadcasted_iota(jnp.int32, (RCH, E), 1)
    oh = (ids == cols).astype(jnp.int32)
    c = oh
    d = 1
    while d < RCH:
        c = c + jnp.concatenate(
            [jnp.zeros((d, E), jnp.int32), c[:RCH - d]], axis=0)
        d *= 2
    base = carry_ref[...] + poff_ref[...] - 1
    dest_ref[...] = jnp.sum((c + base) * oh, axis=1, keepdims=True)
    carry_ref[...] = carry_ref[...] + c[RCH - 1:RCH, :]


def _pair_dests(flat_e, poff):
    return pl.pallas_call(
        _rank_body,
        grid=(NRCH,),
        in_specs=[pl.BlockSpec((RCH, 1), lambda i: (i, 0)),
                  pl.BlockSpec((1, E), lambda i: (0, 0))],
        out_specs=pl.BlockSpec((RCH, 1), lambda i: (i, 0)),
        out_shape=jax.ShapeDtypeStruct((T * K, 1), jnp.int32),
        scratch_shapes=[pltpu.VMEM((1, E), jnp.int32)],
        compiler_params=pltpu.CompilerParams(
            dimension_semantics=("arbitrary",)),
    )(flat_e.reshape(T * K, 1), poff.reshape(1, E))


def _sc_worker_id():
    return lax.axis_index("s") * NC + lax.axis_index("c")


@functools.partial(
    pl.kernel,
    out_type=(jax.ShapeDtypeStruct((R, H), jnp.float32),
              jax.ShapeDtypeStruct((R,), jnp.float32)),
    mesh=plsc.VectorSubcoreMesh(core_axis_name="c", subcore_axis_name="s"),
    scratch_types=[
        pltpu.VMEM((GCH,), jnp.int32),
        pltpu.VMEM((GCH,), jnp.int32),
        pltpu.VMEM((GCH,), jnp.float32),
        pltpu.VMEM((GCH,), jnp.float32),
        pltpu.VMEM((GCH, H), jnp.float32),
        pltpu.SemaphoreType.DMA,
        pltpu.SemaphoreType.DMA,
        pltpu.SemaphoreType.DMA,
        pltpu.SemaphoreType.DMA,
    ],
)
def _sc_dispatch(hid_hbm, d0_hbm, d1_hbm, w0_hbm, w1_hbm,
                 xs_hbm, ws_hbm, i0_v, i1_v, w0_v, w1_v, buf_v,
                 sem0, sem1, sem2, sem3):
    base = _sc_worker_id() * (T // NW)
    def body(i, carry):
        b = base + i * GCH
        pltpu.sync_copy(d0_hbm.at[pl.ds(b, GCH)], i0_v)
        pltpu.sync_copy(d1_hbm.at[pl.ds(b, GCH)], i1_v)
        pltpu.sync_copy(w0_hbm.at[pl.ds(b, GCH)], w0_v)
        pltpu.sync_copy(w1_hbm.at[pl.ds(b, GCH)], w1_v)
        pltpu.sync_copy(hid_hbm.at[pl.ds(b, GCH)], buf_v)
        c0 = pltpu.async_copy(buf_v, xs_hbm.at[i0_v], sem0)
        c1 = pltpu.async_copy(buf_v, xs_hbm.at[i1_v], sem1)
        c2 = pltpu.async_copy(w0_v, ws_hbm.at[i0_v], sem2)
        c3 = pltpu.async_copy(w1_v, ws_hbm.at[i1_v], sem3)
        c0.wait()
        c1.wait()
        c2.wait()
        c3.wait()
        return carry
    lax.fori_loop(0, (T // NW) // GCH, body, 0)


def _gemm_body(be_ref, xs_ref, ew_ref, eb_ref, ws_ref, out_ref):
    acc = lax.dot_general(xs_ref[...].astype(jnp.bfloat16),
                          ew_ref[0].astype(jnp.bfloat16),
                          (((1,), (1,)), ((), ())),
                          preferred_element_type=jnp.float32)
    out_ref[...] = (acc + eb_ref[0]) * ws_ref[...]


def _grouped_gemm(x_sorted, expert_w, expert_b, w_sorted, blk_e):
    grid_spec = pltpu.PrefetchScalarGridSpec(
        num_scalar_prefetch=1,
        grid=(NBLK,),
        in_specs=[
            pl.BlockSpec((BM, H), lambda i, be: (i, 0)),
            pl.BlockSpec((1, H, H), lambda i, be: (be[i], 0, 0)),
            pl.BlockSpec((1, 1, H), lambda i, be: (be[i], 0, 0)),
            pl.BlockSpec((BM, 1), lambda i, be: (i, 0)),
        ],
        out_specs=pl.BlockSpec((BM, H), lambda i, be: (i, 0)),
    )
    return pl.pallas_call(
        _gemm_body,
        grid_spec=grid_spec,
        out_shape=jax.ShapeDtypeStruct((R, H), jnp.float32),
    )(blk_e, x_sorted, expert_w, expert_b.reshape(E, 1, H),
      w_sorted.reshape(R, 1))


@functools.partial(
    pl.kernel,
    out_type=(jax.ShapeDtypeStruct((T, H), jnp.float32),
              jax.ShapeDtypeStruct((T, H), jnp.float32)),
    mesh=plsc.VectorSubcoreMesh(core_axis_name="c", subcore_axis_name="s"),
    scratch_types=[
        pltpu.VMEM((GCH,), jnp.int32),
        pltpu.VMEM((GCH,), jnp.int32),
        pltpu.VMEM((GCH, H), jnp.float32),
        pltpu.VMEM((GCH, H), jnp.float32),
        pltpu.SemaphoreType.DMA,
        pltpu.SemaphoreType.DMA,
    ],
)
def _sc_gather_pairs(outs_hbm, p0_hbm, p1_hbm, g0_hbm, g1_hbm,
                     i0_v, i1_v, b0_v, b1_v, sem0, sem1):
    base = _sc_worker_id() * (T // NW)
    def body(i, carry):
        b = base + i * GCH
        pltpu.sync_copy(p0_hbm.at[pl.ds(b, GCH)], i0_v)
        pltpu.sync_copy(p1_hbm.at[pl.ds(b, GCH)], i1_v)
        c0 = pltpu.async_copy(outs_hbm.at[i0_v], b0_v, sem0)
        c1 = pltpu.async_copy(outs_hbm.at[i1_v], b1_v, sem1)
        c0.wait()
        c1.wait()
        pltpu.sync_copy(b0_v, g0_hbm.at[pl.ds(b, GCH)])
        pltpu.sync_copy(b1_v, g1_hbm.at[pl.ds(b, GCH)])
        return carry
    lax.fori_loop(0, (T // NW) // GCH, body, 0)


def _add_body(a_ref, b_ref, o_ref):
    o_ref[...] = a_ref[...] + b_ref[...]


def _combine_add(g0, g1):
    return pl.pallas_call(
        _add_body,
        grid=(T // BT,),
        in_specs=[pl.BlockSpec((BT, H), lambda i: (i, 0)),
                  pl.BlockSpec((BT, H), lambda i: (i, 0))],
        out_specs=pl.BlockSpec((BT, H), lambda i: (i, 0)),
        out_shape=jax.ShapeDtypeStruct((T, H), jnp.float32),
    )(g0, g1)


def kernel(x, gate_w, gate_b, expert_w, expert_b):
    hidden = x.reshape(T, H)
    logits, top_idx, top_w = _router(hidden, gate_w, gate_b)

    flat_e = top_idx.reshape(-1)
    counts = jnp.zeros((E,), jnp.int32).at[flat_e].add(1)
    padded = ((counts + BM - 1) // BM) * BM
    cpad = jnp.cumsum(padded)
    poff = cpad - padded
    d2 = _pair_dests(flat_e, poff).reshape(T, K)
    p0 = d2[:, 0]
    p1 = d2[:, 1]
    w0 = top_w[:, 0]
    w1 = top_w[:, 1]
    blk_start = jnp.arange(NBLK, dtype=jnp.int32) * BM
    blk_e = jnp.minimum(
        jnp.sum((blk_start[:, None] >= cpad[None, :]).astype(jnp.int32),
                axis=1),
        E - 1).astype(jnp.int32)

    x_sorted, w_sorted = _sc_dispatch(hidden, p0, p1, w0, w1)
    out_sorted = _grouped_gemm(x_sorted, expert_w, expert_b, w_sorted, blk_e)
    g0, g1 = _sc_gather_pairs(out_sorted, p0, p1)
    final = _combine_add(g0, g1)
    return final.reshape(B, S, H), logits


# speedup vs baseline: 2.4937x; 1.0914x over previous
import functools

import jax
import jax.numpy as jnp
from jax import lax
from jax.experimental import pallas as pl
from jax.experimental.pallas import tpu as pltpu
from jax.experimental.pallas import tpu_sc as plsc

B, S, H, E, K = 4, 4096, 768, 64, 2
T = B * S
BM = 128
R = T * K + E * BM
NBLK = R // BM
BT = 1024

NC, NS = 2, 16
NW = NC * NS
GCH = 64


def _router_body(x_ref, gwt_ref, gb_ref, logits_ref, idx_ref, w_ref):
    xb = x_ref[...]
    logits = lax.dot_general(xb, gwt_ref[...], (((1,), (0,)), ((), ())),
                             preferred_element_type=jnp.float32)
    logits = logits + gb_ref[...]
    logits_ref[...] = logits
    ids = lax.broadcasted_iota(jnp.int32, logits.shape, 1)
    m1 = jnp.max(logits, axis=1, keepdims=True)
    i1 = jnp.min(jnp.where(logits == m1, ids, E), axis=1, keepdims=True)
    masked = jnp.where(ids == i1, -jnp.inf, logits)
    m2 = jnp.max(masked, axis=1, keepdims=True)
    i2 = jnp.min(jnp.where(masked == m2, ids, E), axis=1, keepdims=True)
    w1 = 1.0 / (1.0 + jnp.exp(m2 - m1))
    idx_ref[...] = jnp.concatenate([i1, i2], axis=1)
    w_ref[...] = jnp.concatenate([w1, 1.0 - w1], axis=1)


def _router(hidden, gate_w, gate_b):
    return pl.pallas_call(
        _router_body,
        grid=(T // BT,),
        in_specs=[
            pl.BlockSpec((BT, H), lambda i: (i, 0)),
            pl.BlockSpec((H, E), lambda i: (0, 0)),
            pl.BlockSpec((1, E), lambda i: (0, 0)),
        ],
        out_specs=[
            pl.BlockSpec((BT, E), lambda i: (i, 0)),
            pl.BlockSpec((BT, K), lambda i: (i, 0)),
            pl.BlockSpec((BT, K), lambda i: (i, 0)),
        ],
        out_shape=[
            jax.ShapeDtypeStruct((T, E), jnp.float32),
            jax.ShapeDtypeStruct((T, K), jnp.int32),
            jax.ShapeDtypeStruct((T, K), jnp.float32),
        ],
    )(hidden, gate_w.T, gate_b.reshape(1, E))


RCH = 2048
NRCH = (T * K) // RCH


def _rank_body(ids_ref, poff_ref, dest_ref, carry_ref):
    @pl.when(pl.program_id(0) == 0)
    def _():
        carry_ref[...] = jnp.zeros_like(carry_ref)
    ids = ids_ref[...]
    cols = lax.broadcasted_iota(jnp.int32, (RCH, E), 1)
    oh = (ids == cols).astype(jnp.int32)
    c = oh
    d = 1
    while d < RCH:
        c = c + jnp.concatenate(
            [jnp.zeros((d, E), jnp.int32), c[:RCH - d]], axis=0)
        d *= 2
    base = carry_ref[...] + poff_ref[...] - 1
    dest_ref[...] = jnp.sum((c + base) * oh, axis=1, keepdims=True)
    carry_ref[...] = carry_ref[...] + c[RCH - 1:RCH, :]


def _pair_dests(flat_e, poff):
    return pl.pallas_call(
        _rank_body,
        grid=(NRCH,),
        in_specs=[pl.BlockSpec((RCH, 1), lambda i: (i, 0)),
                  pl.BlockSpec((1, E), lambda i: (0, 0))],
        out_specs=pl.BlockSpec((RCH, 1), lambda i: (i, 0)),
        out_shape=jax.ShapeDtypeStruct((T * K, 1), jnp.int32),
        scratch_shapes=[pltpu.VMEM((1, E), jnp.int32)],
        compiler_params=pltpu.CompilerParams(
            dimension_semantics=("arbitrary",)),
    )(flat_e.reshape(T * K, 1), poff.reshape(1, E))


def _sc_worker_id():
    return lax.axis_index("s") * NC + lax.axis_index("c")


@functools.partial(
    pl.kernel,
    out_type=(jax.ShapeDtypeStruct((R, H), jnp.float32),
              jax.ShapeDtypeStruct((R,), jnp.float32)),
    mesh=plsc.VectorSubcoreMesh(core_axis_name="c", subcore_axis_name="s"),
    scratch_types=[
        pltpu.VMEM((GCH,), jnp.int32),
        pltpu.VMEM((GCH,), jnp.int32),
        pltpu.VMEM((GCH,), jnp.float32),
        pltpu.VMEM((GCH,), jnp.float32),
        pltpu.VMEM((GCH, H), jnp.float32),
        pltpu.SemaphoreType.DMA,
        pltpu.SemaphoreType.DMA,
        pltpu.SemaphoreType.DMA,
        pltpu.SemaphoreType.DMA,
    ],
)
def _sc_dispatch(hid_hbm, d0_hbm, d1_hbm, w0_hbm, w1_hbm,
                 xs_hbm, ws_hbm, i0_v, i1_v, w0_v, w1_v, buf_v,
                 sem0, sem1, sem2, sem3):
    base = _sc_worker_id() * (T // NW)
    def body(i, carry):
        b = base + i * GCH
        pltpu.sync_copy(d0_hbm.at[pl.ds(b, GCH)], i0_v)
        pltpu.sync_copy(d1_hbm.at[pl.ds(b, GCH)], i1_v)
        pltpu.sync_copy(w0_hbm.at[pl.ds(b, GCH)], w0_v)
        pltpu.sync_copy(w1_hbm.at[pl.ds(b, GCH)], w1_v)
        pltpu.sync_copy(hid_hbm.at[pl.ds(b, GCH)], buf_v)
        c0 = pltpu.async_copy(buf_v, xs_hbm.at[i0_v], sem0)
        c1 = pltpu.async_copy(buf_v, xs_hbm.at[i1_v], sem1)
        c2 = pltpu.async_copy(w0_v, ws_hbm.at[i0_v], sem2)
        c3 = pltpu.async_copy(w1_v, ws_hbm.at[i1_v], sem3)
        c0.wait()
        c1.wait()
        c2.wait()
        c3.wait()
        return carry
    lax.fori_loop(0, (T // NW) // GCH, body, 0)


def _gemm_body(be_ref, xs_ref, ew_ref, eb_ref, ws_ref, out_ref):
    acc = lax.dot_general(xs_ref[...].astype(jnp.bfloat16),
                          ew_ref[0].astype(jnp.bfloat16),
                          (((1,), (1,)), ((), ())),
                          preferred_element_type=jnp.float32)
    out_ref[...] = (acc + eb_ref[0]) * ws_ref[...]


def _grouped_gemm(x_sorted, expert_w, expert_b, w_sorted, blk_e):
    grid_spec = pltpu.PrefetchScalarGridSpec(
        num_scalar_prefetch=1,
        grid=(NBLK,),
        in_specs=[
            pl.BlockSpec((BM, H), lambda i, be: (i, 0)),
            pl.BlockSpec((1, H, H), lambda i, be: (be[i], 0, 0)),
            pl.BlockSpec((1, 1, H), lambda i, be: (be[i], 0, 0)),
            pl.BlockSpec((BM, 1), lambda i, be: (i, 0)),
        ],
        out_specs=pl.BlockSpec((BM, H), lambda i, be: (i, 0)),
    )
    return pl.pallas_call(
        _gemm_body,
        grid_spec=grid_spec,
        out_shape=jax.ShapeDtypeStruct((R, H), jnp.float32),
    )(blk_e, x_sorted, expert_w, expert_b.reshape(E, 1, H),
      w_sorted.reshape(R, 1))


@functools.partial(
    pl.kernel,
    out_type=jax.ShapeDtypeStruct((T, H), jnp.float32),
    mesh=plsc.VectorSubcoreMesh(core_axis_name="c", subcore_axis_name="s"),
    scratch_types=[
        pltpu.VMEM((GCH,), jnp.int32),
        pltpu.VMEM((GCH,), jnp.int32),
        pltpu.VMEM((GCH, H), jnp.float32),
        pltpu.SemaphoreType.DMA,
        pltpu.SemaphoreType.DMA,
    ],
)
def _sc_combine(outs_hbm, p0_hbm, p1_hbm, fin_hbm,
                i0_v, i1_v, buf_v, sem0, sem1):
    base = _sc_worker_id() * (T // NW)
    def body(i, carry):
        b = base + i * GCH
        pltpu.sync_copy(p0_hbm.at[pl.ds(b, GCH)], i0_v)
        pltpu.sync_copy(p1_hbm.at[pl.ds(b, GCH)], i1_v)
        pltpu.async_copy(outs_hbm.at[i0_v], buf_v, sem0).wait()
        pltpu.async_copy(outs_hbm.at[i1_v], buf_v, sem1, add=True).wait()
        pltpu.sync_copy(buf_v, fin_hbm.at[pl.ds(b, GCH)])
        return carry
    lax.fori_loop(0, (T // NW) // GCH, body, 0)


def _add_body(a_ref, b_ref, o_ref):
    o_ref[...] = a_ref[...] + b_ref[...]


def _combine_add(g0, g1):
    return pl.pallas_call(
        _add_body,
        grid=(T // BT,),
        in_specs=[pl.BlockSpec((BT, H), lambda i: (i, 0)),
                  pl.BlockSpec((BT, H), lambda i: (i, 0))],
        out_specs=pl.BlockSpec((BT, H), lambda i: (i, 0)),
        out_shape=jax.ShapeDtypeStruct((T, H), jnp.float32),
    )(g0, g1)


def kernel(x, gate_w, gate_b, expert_w, expert_b):
    hidden = x.reshape(T, H)
    logits, top_idx, top_w = _router(hidden, gate_w, gate_b)

    flat_e = top_idx.reshape(-1)
    counts = jnp.zeros((E,), jnp.int32).at[flat_e].add(1)
    padded = ((counts + BM - 1) // BM) * BM
    cpad = jnp.cumsum(padded)
    poff = cpad - padded
    d2 = _pair_dests(flat_e, poff).reshape(T, K)
    p0 = d2[:, 0]
    p1 = d2[:, 1]
    w0 = top_w[:, 0]
    w1 = top_w[:, 1]
    blk_start = jnp.arange(NBLK, dtype=jnp.int32) * BM
    blk_e = jnp.minimum(
        jnp.sum((blk_start[:, None] >= cpad[None, :]).astype(jnp.int32),
                axis=1),
        E - 1).astype(jnp.int32)

    x_sorted, w_sorted = _sc_dispatch(hidden, p0, p1, w0, w1)
    out_sorted = _grouped_gemm(x_sorted, expert_w, expert_b, w_sorted, blk_e)
    final = _sc_combine(out_sorted, p0, p1)
    return final.reshape(B, S, H), logits
